# Initial kernel scaffold; baseline (speedup 1.0000x reference)
import functools

import jax
import jax.numpy as jnp
from jax import lax
from jax.experimental import pallas as pl
from jax.experimental.pallas import tpu as pltpu
from jax.experimental.pallas import tpu_sc as plsc

_N = 10000
_E = 160000
_NPAD = 10240
_D_IN = 256
_D_HID = 128

_NC = 2
_NS = 16
_CB = 125

_DEG_NCHUNK = 40
_AGG_NCHUNK = 80

_sc_mesh = plsc.VectorSubcoreMesh(core_axis_name="c", subcore_axis_name="s")


@functools.partial(
    pl.kernel,
    mesh=_sc_mesh,
    out_type=jax.ShapeDtypeStruct((_NC, _NPAD), jnp.float32),
    scratch_types=[
        pltpu.VMEM((_DEG_NCHUNK, _CB), jnp.int32),
        pltpu.VMEM((128,), jnp.float32),
        pltpu.VMEM((640,), jnp.float32),
        pltpu.VMEM_SHARED((_NPAD,), jnp.float32),
    ],
)
def _deg_kernel(dst_hbm, out_hbm, idx_v, ones_v, zeros_v, acc_sh):
    cid = lax.axis_index("c")
    sid = lax.axis_index("s")
    w = cid * _NS + sid
    for i in range(8):
        ones_v[pl.ds(i * 16, 16)] = jnp.ones((16,), jnp.float32)

    def zfill(k, carry):
        zeros_v[pl.ds(k * 16, 16)] = jnp.zeros((16,), jnp.float32)
        return carry

    lax.fori_loop(0, 40, zfill, 0)
    pltpu.sync_copy(zeros_v, acc_sh.at[pl.ds(sid * 640, 640)])
    pltpu.sync_copy(dst_hbm.at[w], idx_v)
    plsc.subcore_barrier()

    def body(j, carry):
        pltpu.sync_copy(ones_v.at[pl.ds(0, _CB)], acc_sh.at[idx_v.at[j]],
                        add=True)
        return carry

    lax.fori_loop(0, _DEG_NCHUNK, body, 0)
    plsc.subcore_barrier()
    pltpu.sync_copy(acc_sh.at[pl.ds(sid * 640, 640)],
                    out_hbm.at[cid, pl.ds(sid * 640, 640)])


@functools.partial(
    pl.kernel,
    mesh=_sc_mesh,
    out_type=[
        jax.ShapeDtypeStruct((_NPAD, _D_HID), jnp.float32),
        jax.ShapeDtypeStruct((_NPAD, _D_HID), jnp.float32),
    ],
    scratch_types=[
        pltpu.VMEM((_AGG_NCHUNK, _CB), jnp.int32),
        pltpu.VMEM((_AGG_NCHUNK, _CB), jnp.int32),
        pltpu.VMEM((2, _CB, _D_HID), jnp.float32),
        pltpu.VMEM((40, _D_HID), jnp.float32),
        pltpu.VMEM_SHARED((_NPAD, _D_HID), jnp.float32),
        pltpu.SemaphoreType.DMA,
        pltpu.SemaphoreType.DMA,
    ],
)
def _agg_kernel(y0_hbm, y1_hbm, src_hbm, dst_hbm, s0_hbm, s1_hbm,
                idx_s, idx_d, rows_v, zeros_v, acc_sh, sem_a, sem_b):
    cid = lax.axis_index("c")
    sid = lax.axis_index("s")

    def zfill(i, carry):
        for j in range(_D_HID // 16):
            zeros_v[i, pl.ds(j * 16, 16)] = jnp.zeros((16,), jnp.float32)
        return carry

    lax.fori_loop(0, 40, zfill, 0)

    def zcopy(k, carry):
        pltpu.sync_copy(zeros_v, acc_sh.at[pl.ds(sid * 640 + k * 40, 40)])
        return carry

    lax.fori_loop(0, 16, zcopy, 0)
    pltpu.sync_copy(src_hbm.at[sid], idx_s)
    pltpu.sync_copy(dst_hbm.at[sid], idx_d)
    plsc.subcore_barrier()

    def run(y_ref):
        pltpu.async_copy(y_ref.at[idx_s.at[0]], rows_v.at[0], sem_a)

        def body(jj, carry):
            ca = 2 * jj
            cb = 2 * jj + 1
            pltpu.async_copy(y_ref.at[idx_s.at[cb]], rows_v.at[1], sem_b)
            pltpu.make_async_copy(y_ref.at[idx_s.at[ca]], rows_v.at[0],
                                  sem_a).wait()
            pltpu.sync_copy(rows_v.at[0], acc_sh.at[idx_d.at[ca]], add=True)
            nxt = lax.rem(ca + 2, _AGG_NCHUNK)
            pltpu.async_copy(y_ref.at[idx_s.at[nxt]], rows_v.at[0], sem_a)
            pltpu.make_async_copy(y_ref.at[idx_s.at[cb]], rows_v.at[1],
                                  sem_b).wait()
            pltpu.sync_copy(rows_v.at[1], acc_sh.at[idx_d.at[cb]], add=True)
            return carry

        lax.fori_loop(0, _AGG_NCHUNK // 2, body, 0)
        pltpu.make_async_copy(y_ref.at[idx_s.at[0]], rows_v.at[0],
                              sem_a).wait()

    @pl.when(cid == 0)
    def _():
        run(y0_hbm)

    @pl.when(cid == 1)
    def _():
        run(y1_hbm)

    plsc.subcore_barrier()

    @pl.when(cid == 0)
    def _():
        pltpu.sync_copy(acc_sh.at[pl.ds(sid * 640, 640)],
                        s0_hbm.at[pl.ds(sid * 640, 640)])

    @pl.when(cid == 1)
    def _():
        pltpu.sync_copy(acc_sh.at[pl.ds(sid * 640, 640)],
                        s1_hbm.at[pl.ds(sid * 640, 640)])


def _prep_body(cnt_ref, x_ref, y0_ref, y1_ref):
    deg = cnt_ref[:, 0] + cnt_ref[:, 1] + 1.0
    dinv = lax.rsqrt(deg)
    y0_ref[...] = x_ref[:, :_D_HID] * dinv[:, None]
    y1_ref[...] = x_ref[:, _D_HID:] * dinv[:, None]


_ROWB = 1000


def _prep_call(cnt_t, x):
    return pl.pallas_call(
        _prep_body,
        grid=(_N // _ROWB,),
        in_specs=[
            pl.BlockSpec((_ROWB, 2), lambda i: (i, 0)),
            pl.BlockSpec((_ROWB, _D_IN), lambda i: (i, 0)),
        ],
        out_specs=[
            pl.BlockSpec((_ROWB, _D_HID), lambda i: (i, 0)),
            pl.BlockSpec((_ROWB, _D_HID), lambda i: (i, 0)),
        ],
        out_shape=[
            jax.ShapeDtypeStruct((_N, _D_HID), jnp.float32),
            jax.ShapeDtypeStruct((_N, _D_HID), jnp.float32),
        ],
    )(cnt_t, x)


def _final_body(cnt_ref, x_ref, s0_ref, s1_ref, a_ref, wcat_ref, bcat_ref,
                wf_ref, bf_ref, wc_ref, bc_ref, out_ref):
    deg = cnt_ref[:, 0] + cnt_ref[:, 1] + 1.0
    dinv = lax.rsqrt(deg)[:, None]
    idg = (1.0 / deg)[:, None]
    agg = jnp.concatenate(
        [s0_ref[...] * dinv + x_ref[:, :_D_HID] * idg,
         s1_ref[...] * dinv + x_ref[:, _D_HID:] * idg], axis=1)
    hcat = jnp.maximum(
        jnp.dot(agg, wcat_ref[...], preferred_element_type=jnp.float32)
        + bcat_ref[...], 0.0)
    alpha = 1.0 - a_ref[...]
    h_fused = alpha * hcat[:, :_D_HID] + (1.0 - alpha) * hcat[:, _D_HID:]
    h = jnp.maximum(
        jnp.dot(h_fused, wf_ref[...], preferred_element_type=jnp.float32)
        + bf_ref[...], 0.0)
    out_ref[...] = (
        jnp.dot(h, wc_ref[...], preferred_element_type=jnp.float32)
        + bc_ref[...])


def _final_call(cnt_t, x, s0, s1, h_node2, wcat, bcat, wf, bf, wc, bc):
    nb = _N // _ROWB
    return pl.pallas_call(
        _final_body,
        grid=(nb,),
        in_specs=[
            pl.BlockSpec((_ROWB, 2), lambda i: (i, 0)),
            pl.BlockSpec((_ROWB, _D_IN), lambda i: (i, 0)),
            pl.BlockSpec((_ROWB, _D_HID), lambda i: (i, 0)),
            pl.BlockSpec((_ROWB, _D_HID), lambda i: (i, 0)),
            pl.BlockSpec((_ROWB, 1), lambda i: (i, 0)),
            pl.BlockSpec((_D_IN, _D_IN), lambda i: (0, 0)),
            pl.BlockSpec((1, _D_IN), lambda i: (0, 0)),
            pl.BlockSpec((_D_HID, 64), lambda i: (0, 0)),
            pl.BlockSpec((1, 64), lambda i: (0, 0)),
            pl.BlockSpec((64, 2), lambda i: (0, 0)),
            pl.BlockSpec((1, 2), lambda i: (0, 0)),
        ],
        out_specs=pl.BlockSpec((_ROWB, 2), lambda i: (i, 0)),
        out_shape=jax.ShapeDtypeStruct((_N, 2), jnp.float32),
    )(cnt_t, x, s0, s1, h_node2, wcat, bcat, wf, bf, wc, bc)


def kernel(x, edge_index, h_node, W_homo, b_homo, W_hetero, b_hetero,
           W_fusion, b_fusion, W_cls, b_cls):
    src = edge_index[0]
    dst = edge_index[1]
    dst32 = dst.reshape(_NC * _NS, _DEG_NCHUNK, _CB)
    cnt = _deg_kernel(dst32)
    cnt_t = cnt.T[:_N]
    y0, y1 = _prep_call(cnt_t, x)
    src16 = src.reshape(_NS, _AGG_NCHUNK, _CB)
    dst16 = dst.reshape(_NS, _AGG_NCHUNK, _CB)
    s0, s1 = _agg_kernel(y0, y1, src16, dst16)
    wcat = jnp.concatenate([W_homo, W_hetero], axis=1)
    bcat = jnp.concatenate([b_homo, b_hetero]).reshape(1, _D_IN)
    logits = _final_call(cnt_t, x, s0[:_N], s1[:_N], h_node.reshape(_N, 1),
                         wcat, bcat, W_fusion, b_fusion.reshape(1, 64),
                         W_cls, b_cls.reshape(1, 2))
    return logits


# speedup vs baseline: 21.3771x; 21.3771x over previous
import functools

import jax
import jax.numpy as jnp
from jax import lax
from jax.experimental import pallas as pl
from jax.experimental.pallas import tpu as pltpu
from jax.experimental.pallas import tpu_sc as plsc

_N = 10000
_E = 160000
_NPAD = 10240
_D_IN = 256
_D_HID = 128

_NC = 2
_NS = 16
_CB = 125

_DEG_NCHUNK = 40
_AGG_NCHUNK = 80
_AGG_STAGES = 5
_AGG_STCH = _AGG_NCHUNK // _AGG_STAGES

_sc_mesh = plsc.VectorSubcoreMesh(core_axis_name="c", subcore_axis_name="s")


@functools.partial(
    pl.kernel,
    mesh=_sc_mesh,
    out_type=jax.ShapeDtypeStruct((_NC, _NPAD), jnp.float32),
    scratch_types=[
        pltpu.VMEM((_DEG_NCHUNK, _CB), jnp.int32),
        pltpu.VMEM((128,), jnp.float32),
        pltpu.VMEM((640,), jnp.float32),
        pltpu.VMEM_SHARED((_NPAD,), jnp.float32),
    ],
)
def _deg_kernel(dst_hbm, out_hbm, idx_v, ones_v, zeros_v, acc_sh):
    cid = lax.axis_index("c")
    sid = lax.axis_index("s")
    w = cid * _NS + sid
    for i in range(8):
        ones_v[pl.ds(i * 16, 16)] = jnp.ones((16,), jnp.float32)

    def zfill(k, carry):
        zeros_v[pl.ds(k * 16, 16)] = jnp.zeros((16,), jnp.float32)
        return carry

    lax.fori_loop(0, 40, zfill, 0)
    pltpu.sync_copy(zeros_v, acc_sh.at[pl.ds(sid * 640, 640)])
    pltpu.sync_copy(dst_hbm.at[w], idx_v)
    plsc.subcore_barrier()

    def body(j, carry):
        pltpu.sync_copy(ones_v.at[pl.ds(0, _CB)], acc_sh.at[idx_v.at[j]],
                        add=True)
        return carry

    lax.fori_loop(0, _DEG_NCHUNK, body, 0)
    plsc.subcore_barrier()
    pltpu.sync_copy(acc_sh.at[pl.ds(sid * 640, 640)],
                    out_hbm.at[cid, pl.ds(sid * 640, 640)])


@functools.partial(
    pl.kernel,
    mesh=_sc_mesh,
    out_type=[
        jax.ShapeDtypeStruct((_NPAD, _D_HID), jnp.float32),
        jax.ShapeDtypeStruct((_NPAD, _D_HID), jnp.float32),
    ],
    scratch_types=[
        pltpu.VMEM((_AGG_STCH, _CB), jnp.int32),
        pltpu.VMEM((_AGG_STCH, _CB), jnp.int32),
        pltpu.VMEM((2, 128, _D_HID), jnp.float32),
        pltpu.VMEM_SHARED((_NPAD, _D_HID), jnp.float32),
        pltpu.SemaphoreType.DMA,
        pltpu.SemaphoreType.DMA,
    ],
)
def _agg_kernel(y0_hbm, y1_hbm, src_hbm, dst_hbm, s0_hbm, s1_hbm,
                idx_s, idx_d, rows_v, acc_sh, sem_a, sem_b):
    cid = lax.axis_index("c")
    sid = lax.axis_index("s")

    def zfill(i, carry):
        for j in range(_D_HID // 16):
            rows_v[0, i, pl.ds(j * 16, 16)] = jnp.zeros((16,), jnp.float32)
        return carry

    lax.fori_loop(0, 128, zfill, 0)

    def zcopy(k, carry):
        pltpu.sync_copy(rows_v.at[0],
                        acc_sh.at[pl.ds(sid * 640 + k * 128, 128)])
        return carry

    lax.fori_loop(0, 5, zcopy, 0)
    plsc.subcore_barrier()

    def run(y_ref):
        buf = [rows_v.at[0, pl.ds(0, _CB)], rows_v.at[1, pl.ds(0, _CB)]]

        def stage(st, carry):
            pltpu.sync_copy(src_hbm.at[sid, pl.ds(st * _AGG_STCH, _AGG_STCH)],
                            idx_s)
            pltpu.sync_copy(dst_hbm.at[sid, pl.ds(st * _AGG_STCH, _AGG_STCH)],
                            idx_d)
            pltpu.async_copy(y_ref.at[idx_s.at[0]], buf[0], sem_a)

            def body(jj, c2):
                ca = 2 * jj
                cb = 2 * jj + 1
                pltpu.async_copy(y_ref.at[idx_s.at[cb]], buf[1], sem_b)
                pltpu.make_async_copy(y_ref.at[idx_s.at[ca]], buf[0],
                                      sem_a).wait()
                pltpu.sync_copy(buf[0], acc_sh.at[idx_d.at[ca]], add=True)
                nxt = lax.rem(ca + 2, _AGG_STCH)
                pltpu.async_copy(y_ref.at[idx_s.at[nxt]], buf[0], sem_a)
                pltpu.make_async_copy(y_ref.at[idx_s.at[cb]], buf[1],
                                      sem_b).wait()
                pltpu.sync_copy(buf[1], acc_sh.at[idx_d.at[cb]], add=True)
                return c2

            lax.fori_loop(0, _AGG_STCH // 2, body, 0)
            pltpu.make_async_copy(y_ref.at[idx_s.at[0]], buf[0],
                                  sem_a).wait()
            return carry

        lax.fori_loop(0, _AGG_STAGES, stage, 0)

    @pl.when(cid == 0)
    def _():
        run(y0_hbm)

    @pl.when(cid == 1)
    def _():
        run(y1_hbm)

    plsc.subcore_barrier()

    @pl.when(cid == 0)
    def _():
        pltpu.sync_copy(acc_sh.at[pl.ds(sid * 640, 640)],
                        s0_hbm.at[pl.ds(sid * 640, 640)])

    @pl.when(cid == 1)
    def _():
        pltpu.sync_copy(acc_sh.at[pl.ds(sid * 640, 640)],
                        s1_hbm.at[pl.ds(sid * 640, 640)])


def _prep_body(cnt_ref, x_ref, y0_ref, y1_ref):
    deg = cnt_ref[:, 0] + cnt_ref[:, 1] + 1.0
    dinv = lax.rsqrt(deg)
    y0_ref[...] = x_ref[:, :_D_HID] * dinv[:, None]
    y1_ref[...] = x_ref[:, _D_HID:] * dinv[:, None]


_ROWB = 1000


def _prep_call(cnt_t, x):
    return pl.pallas_call(
        _prep_body,
        grid=(_N // _ROWB,),
        in_specs=[
            pl.BlockSpec((_ROWB, 2), lambda i: (i, 0)),
            pl.BlockSpec((_ROWB, _D_IN), lambda i: (i, 0)),
        ],
        out_specs=[
            pl.BlockSpec((_ROWB, _D_HID), lambda i: (i, 0)),
            pl.BlockSpec((_ROWB, _D_HID), lambda i: (i, 0)),
        ],
        out_shape=[
            jax.ShapeDtypeStruct((_N, _D_HID), jnp.float32),
            jax.ShapeDtypeStruct((_N, _D_HID), jnp.float32),
        ],
    )(cnt_t, x)


def _final_body(cnt_ref, x_ref, s0_ref, s1_ref, a_ref, wcat_ref, bcat_ref,
                wf_ref, bf_ref, wc_ref, bc_ref, out_ref):
    deg = cnt_ref[:, 0] + cnt_ref[:, 1] + 1.0
    dinv = lax.rsqrt(deg)[:, None]
    idg = (1.0 / deg)[:, None]
    agg = jnp.concatenate(
        [s0_ref[...] * dinv + x_ref[:, :_D_HID] * idg,
         s1_ref[...] * dinv + x_ref[:, _D_HID:] * idg], axis=1)
    hcat = jnp.maximum(
        jnp.dot(agg, wcat_ref[...], preferred_element_type=jnp.float32,
                precision=lax.Precision.HIGHEST)
        + bcat_ref[...], 0.0)
    alpha = 1.0 - a_ref[...]
    h_fused = alpha * hcat[:, :_D_HID] + (1.0 - alpha) * hcat[:, _D_HID:]
    h = jnp.maximum(
        jnp.dot(h_fused, wf_ref[...], preferred_element_type=jnp.float32,
                precision=lax.Precision.HIGHEST)
        + bf_ref[...], 0.0)
    out_ref[...] = (
        jnp.dot(h, wc_ref[...], preferred_element_type=jnp.float32,
                precision=lax.Precision.HIGHEST)
        + bc_ref[...])


def _final_call(cnt_t, x, s0, s1, h_node2, wcat, bcat, wf, bf, wc, bc):
    nb = _N // _ROWB
    return pl.pallas_call(
        _final_body,
        grid=(nb,),
        in_specs=[
            pl.BlockSpec((_ROWB, 2), lambda i: (i, 0)),
            pl.BlockSpec((_ROWB, _D_IN), lambda i: (i, 0)),
            pl.BlockSpec((_ROWB, _D_HID), lambda i: (i, 0)),
            pl.BlockSpec((_ROWB, _D_HID), lambda i: (i, 0)),
            pl.BlockSpec((_ROWB, 1), lambda i: (i, 0)),
            pl.BlockSpec((_D_IN, _D_IN), lambda i: (0, 0)),
            pl.BlockSpec((1, _D_IN), lambda i: (0, 0)),
            pl.BlockSpec((_D_HID, 64), lambda i: (0, 0)),
            pl.BlockSpec((1, 64), lambda i: (0, 0)),
            pl.BlockSpec((64, 2), lambda i: (0, 0)),
            pl.BlockSpec((1, 2), lambda i: (0, 0)),
        ],
        out_specs=pl.BlockSpec((_ROWB, 2), lambda i: (i, 0)),
        out_shape=jax.ShapeDtypeStruct((_N, 2), jnp.float32),
    )(cnt_t, x, s0, s1, h_node2, wcat, bcat, wf, bf, wc, bc)


def kernel(x, edge_index, h_node, W_homo, b_homo, W_hetero, b_hetero,
           W_fusion, b_fusion, W_cls, b_cls):
    src = edge_index[0]
    dst = edge_index[1]
    dst32 = dst.reshape(_NC * _NS, _DEG_NCHUNK, _CB)
    cnt = _deg_kernel(dst32)
    cnt_t = cnt.T[:_N]
    y0, y1 = _prep_call(cnt_t, x)
    src16 = src.reshape(_NS, _AGG_NCHUNK, _CB)
    dst16 = dst.reshape(_NS, _AGG_NCHUNK, _CB)
    s0, s1 = _agg_kernel(y0, y1, src16, dst16)
    wcat = jnp.concatenate([W_homo, W_hetero], axis=1)
    bcat = jnp.concatenate([b_homo, b_hetero]).reshape(1, _D_IN)
    logits = _final_call(cnt_t, x, s0[:_N], s1[:_N], h_node.reshape(_N, 1),
                         wcat, bcat, W_fusion, b_fusion.reshape(1, 64),
                         W_cls, b_cls.reshape(1, 2))
    return logits


# speedup vs baseline: 25.9444x; 1.2137x over previous
import functools

import jax
import jax.numpy as jnp
from jax import lax
from jax.experimental import pallas as pl
from jax.experimental.pallas import tpu as pltpu
from jax.experimental.pallas import tpu_sc as plsc

_N = 10000
_E = 160000
_NPAD = 10240
_D_IN = 256
_D_HID = 128

_NC = 2
_NS = 16
_CB = 125

_DEG_NCHUNK = 40
_AGG_NCHUNK = 80
_AGG_STAGES = 5
_AGG_STCH = _AGG_NCHUNK // _AGG_STAGES

_sc_mesh = plsc.VectorSubcoreMesh(core_axis_name="c", subcore_axis_name="s")


@functools.partial(
    pl.kernel,
    mesh=_sc_mesh,
    out_type=jax.ShapeDtypeStruct((_NC, _NPAD), jnp.float32),
    scratch_types=[
        pltpu.VMEM((_DEG_NCHUNK, _CB), jnp.int32),
        pltpu.VMEM((128,), jnp.float32),
        pltpu.VMEM((640,), jnp.float32),
        pltpu.VMEM_SHARED((_NPAD,), jnp.float32),
    ],
)
def _deg_kernel(edge_hbm, out_hbm, idx_v, ones_v, zeros_v, acc_sh):
    cid = lax.axis_index("c")
    sid = lax.axis_index("s")
    w = cid * _NS + sid
    for i in range(8):
        ones_v[pl.ds(i * 16, 16)] = jnp.ones((16,), jnp.float32)

    def zfill(k, carry):
        zeros_v[pl.ds(k * 16, 16)] = jnp.zeros((16,), jnp.float32)
        return carry

    lax.fori_loop(0, 40, zfill, 0)
    pltpu.sync_copy(zeros_v, acc_sh.at[pl.ds(sid * 640, 640)])
    pltpu.sync_copy(
        edge_hbm.at[1, w // 2, pl.ds((w % 2) * _DEG_NCHUNK, _DEG_NCHUNK)],
        idx_v)
    plsc.subcore_barrier()

    def body(j, carry):
        pltpu.sync_copy(ones_v.at[pl.ds(0, _CB)], acc_sh.at[idx_v.at[j]],
                        add=True)
        return carry

    lax.fori_loop(0, _DEG_NCHUNK, body, 0)
    plsc.subcore_barrier()
    pltpu.sync_copy(acc_sh.at[pl.ds(sid * 640, 640)],
                    out_hbm.at[cid, pl.ds(sid * 640, 640)])


@functools.partial(
    pl.kernel,
    mesh=_sc_mesh,
    out_type=[
        jax.ShapeDtypeStruct((_NPAD, _D_HID), jnp.float32),
        jax.ShapeDtypeStruct((_NPAD, _D_HID), jnp.float32),
    ],
    scratch_types=[
        pltpu.VMEM((_AGG_STCH, _CB), jnp.int32),
        pltpu.VMEM((_AGG_STCH, _CB), jnp.int32),
        pltpu.VMEM((2, 128, _D_HID), jnp.float32),
        pltpu.VMEM_SHARED((_NPAD, _D_HID), jnp.float32),
        pltpu.SemaphoreType.DMA,
        pltpu.SemaphoreType.DMA,
    ],
)
def _agg_kernel(y0_hbm, y1_hbm, edge_hbm, s0_hbm, s1_hbm,
                idx_s, idx_d, rows_v, acc_sh, sem_a, sem_b):
    cid = lax.axis_index("c")
    sid = lax.axis_index("s")

    def zfill(i, carry):
        for j in range(_D_HID // 16):
            rows_v[0, i, pl.ds(j * 16, 16)] = jnp.zeros((16,), jnp.float32)
        return carry

    lax.fori_loop(0, 128, zfill, 0)

    def zcopy(k, carry):
        pltpu.sync_copy(rows_v.at[0],
                        acc_sh.at[pl.ds(sid * 640 + k * 128, 128)])
        return carry

    lax.fori_loop(0, 5, zcopy, 0)
    plsc.subcore_barrier()

    def run(y_ref):
        buf = [rows_v.at[0, pl.ds(0, _CB)], rows_v.at[1, pl.ds(0, _CB)]]

        def stage(st, carry):
            pltpu.sync_copy(
                edge_hbm.at[0, sid, pl.ds(st * _AGG_STCH, _AGG_STCH)], idx_s)
            pltpu.sync_copy(
                edge_hbm.at[1, sid, pl.ds(st * _AGG_STCH, _AGG_STCH)], idx_d)
            pltpu.async_copy(y_ref.at[idx_s.at[0]], buf[0], sem_a)

            def body(jj, c2):
                ca = 2 * jj
                cb = 2 * jj + 1
                pltpu.async_copy(y_ref.at[idx_s.at[cb]], buf[1], sem_b)
                pltpu.make_async_copy(y_ref.at[idx_s.at[ca]], buf[0],
                                      sem_a).wait()
                pltpu.sync_copy(buf[0], acc_sh.at[idx_d.at[ca]], add=True)
                nxt = lax.rem(ca + 2, _AGG_STCH)
                pltpu.async_copy(y_ref.at[idx_s.at[nxt]], buf[0], sem_a)
                pltpu.make_async_copy(y_ref.at[idx_s.at[cb]], buf[1],
                                      sem_b).wait()
                pltpu.sync_copy(buf[1], acc_sh.at[idx_d.at[cb]], add=True)
                return c2

            lax.fori_loop(0, _AGG_STCH // 2, body, 0)
            pltpu.make_async_copy(y_ref.at[idx_s.at[0]], buf[0],
                                  sem_a).wait()
            return carry

        lax.fori_loop(0, _AGG_STAGES, stage, 0)

    @pl.when(cid == 0)
    def _():
        run(y0_hbm)

    @pl.when(cid == 1)
    def _():
        run(y1_hbm)

    plsc.subcore_barrier()

    @pl.when(cid == 0)
    def _():
        pltpu.sync_copy(acc_sh.at[pl.ds(sid * 640, 640)],
                        s0_hbm.at[pl.ds(sid * 640, 640)])

    @pl.when(cid == 1)
    def _():
        pltpu.sync_copy(acc_sh.at[pl.ds(sid * 640, 640)],
                        s1_hbm.at[pl.ds(sid * 640, 640)])


def _prep_body(cnt_ref, x_ref, y0_ref, y1_ref):
    deg = cnt_ref[:, 0] + cnt_ref[:, 1] + 1.0
    dinv = lax.rsqrt(deg)
    y0_ref[...] = x_ref[:, :_D_HID] * dinv[:, None]
    y1_ref[...] = x_ref[:, _D_HID:] * dinv[:, None]


_ROWB = 1000


def _prep_call(cnt_t, x):
    return pl.pallas_call(
        _prep_body,
        grid=(_N // _ROWB,),
        in_specs=[
            pl.BlockSpec((_ROWB, 2), lambda i: (i, 0)),
            pl.BlockSpec((_ROWB, _D_IN), lambda i: (i, 0)),
        ],
        out_specs=[
            pl.BlockSpec((_ROWB, _D_HID), lambda i: (i, 0)),
            pl.BlockSpec((_ROWB, _D_HID), lambda i: (i, 0)),
        ],
        out_shape=[
            jax.ShapeDtypeStruct((_N, _D_HID), jnp.float32),
            jax.ShapeDtypeStruct((_N, _D_HID), jnp.float32),
        ],
    )(cnt_t, x)


def _final_body(cnt_ref, x_ref, s0_ref, s1_ref, a_ref, wcat_ref, bcat_ref,
                wf_ref, bf_ref, wc_ref, bc_ref, out_ref):
    deg = cnt_ref[:, 0] + cnt_ref[:, 1] + 1.0
    dinv = lax.rsqrt(deg)[:, None]
    idg = (1.0 / deg)[:, None]
    agg = jnp.concatenate(
        [s0_ref[...] * dinv + x_ref[:, :_D_HID] * idg,
         s1_ref[...] * dinv + x_ref[:, _D_HID:] * idg], axis=1)
    hcat = jnp.maximum(
        jnp.dot(agg, wcat_ref[...], preferred_element_type=jnp.float32)
        + bcat_ref[...], 0.0)
    alpha = 1.0 - a_ref[...]
    h_fused = alpha * hcat[:, :_D_HID] + (1.0 - alpha) * hcat[:, _D_HID:]
    h = jnp.maximum(
        jnp.dot(h_fused, wf_ref[...], preferred_element_type=jnp.float32)
        + bf_ref[...], 0.0)
    out_ref[...] = (
        jnp.dot(h, wc_ref[...], preferred_element_type=jnp.float32)
        + bc_ref[...])


def _final_call(cnt_t, x, s0, s1, h_node2, wcat, bcat, wf, bf, wc, bc):
    nb = _N // _ROWB
    return pl.pallas_call(
        _final_body,
        grid=(nb,),
        in_specs=[
            pl.BlockSpec((_ROWB, 2), lambda i: (i, 0)),
            pl.BlockSpec((_ROWB, _D_IN), lambda i: (i, 0)),
            pl.BlockSpec((_ROWB, _D_HID), lambda i: (i, 0)),
            pl.BlockSpec((_ROWB, _D_HID), lambda i: (i, 0)),
            pl.BlockSpec((_ROWB, 1), lambda i: (i, 0)),
            pl.BlockSpec((_D_IN, _D_IN), lambda i: (0, 0)),
            pl.BlockSpec((1, _D_IN), lambda i: (0, 0)),
            pl.BlockSpec((_D_HID, 64), lambda i: (0, 0)),
            pl.BlockSpec((1, 64), lambda i: (0, 0)),
            pl.BlockSpec((64, 2), lambda i: (0, 0)),
            pl.BlockSpec((1, 2), lambda i: (0, 0)),
        ],
        out_specs=pl.BlockSpec((_ROWB, 2), lambda i: (i, 0)),
        out_shape=jax.ShapeDtypeStruct((_N, 2), jnp.float32),
    )(cnt_t, x, s0, s1, h_node2, wcat, bcat, wf, bf, wc, bc)


def kernel(x, edge_index, h_node, W_homo, b_homo, W_hetero, b_hetero,
           W_fusion, b_fusion, W_cls, b_cls):
    e4 = edge_index.reshape(2, _NS, _AGG_NCHUNK, _CB)
    cnt = _deg_kernel(e4)
    cnt_t = cnt.T[:_N]
    y0, y1 = _prep_call(cnt_t, x)
    s0, s1 = _agg_kernel(y0, y1, e4)
    wcat = jnp.concatenate([W_homo, W_hetero], axis=1)
    bcat = jnp.concatenate([b_homo, b_hetero]).reshape(1, _D_IN)
    logits = _final_call(cnt_t, x, s0, s1, h_node.reshape(_N, 1),
                         wcat, bcat, W_fusion, b_fusion.reshape(1, 64),
                         W_cls, b_cls.reshape(1, 2))
    return logits


# speedup vs baseline: 26.3810x; 1.0168x over previous
import functools

import jax
import jax.numpy as jnp
from jax import lax
from jax.experimental import pallas as pl
from jax.experimental.pallas import tpu as pltpu
from jax.experimental.pallas import tpu_sc as plsc

_N = 10000
_E = 160000
_NPAD = 10240
_D_IN = 256
_D_HID = 128

_NC = 2
_NS = 16
_CB = 125

_DEG_NCHUNK = 40
_AGG_NCHUNK = 80
_AGG_STAGES = 5
_AGG_STCH = _AGG_NCHUNK // _AGG_STAGES

_sc_mesh = plsc.VectorSubcoreMesh(core_axis_name="c", subcore_axis_name="s")


@functools.partial(
    pl.kernel,
    mesh=_sc_mesh,
    out_type=jax.ShapeDtypeStruct((_NC, _NPAD), jnp.float32),
    scratch_types=[
        pltpu.VMEM((_DEG_NCHUNK, _CB), jnp.int32),
        pltpu.VMEM((128,), jnp.float32),
        pltpu.VMEM((640,), jnp.float32),
        pltpu.VMEM_SHARED((_NPAD,), jnp.float32),
    ],
)
def _deg_kernel(edge_hbm, out_hbm, idx_v, ones_v, zeros_v, acc_sh):
    cid = lax.axis_index("c")
    sid = lax.axis_index("s")
    w = cid * _NS + sid
    for i in range(8):
        ones_v[pl.ds(i * 16, 16)] = jnp.ones((16,), jnp.float32)

    def zfill(k, carry):
        zeros_v[pl.ds(k * 16, 16)] = jnp.zeros((16,), jnp.float32)
        return carry

    lax.fori_loop(0, 40, zfill, 0)
    pltpu.sync_copy(zeros_v, acc_sh.at[pl.ds(sid * 640, 640)])
    pltpu.sync_copy(
        edge_hbm.at[1, w // 2, pl.ds((w % 2) * _DEG_NCHUNK, _DEG_NCHUNK)],
        idx_v)
    plsc.subcore_barrier()

    def body(j, carry):
        pltpu.sync_copy(ones_v.at[pl.ds(0, _CB)], acc_sh.at[idx_v.at[j]],
                        add=True)
        return carry

    lax.fori_loop(0, _DEG_NCHUNK, body, 0)
    plsc.subcore_barrier()
    pltpu.sync_copy(acc_sh.at[pl.ds(sid * 640, 640)],
                    out_hbm.at[cid, pl.ds(sid * 640, 640)])


@functools.partial(
    pl.kernel,
    mesh=_sc_mesh,
    out_type=[
        jax.ShapeDtypeStruct((_NPAD, _D_HID), jnp.float32),
        jax.ShapeDtypeStruct((_NPAD, _D_HID), jnp.float32),
    ],
    scratch_types=[
        pltpu.VMEM((_AGG_STCH, _CB), jnp.int32),
        pltpu.VMEM((_AGG_STCH, _CB), jnp.int32),
        pltpu.VMEM((2, 128, _D_HID), jnp.float32),
        pltpu.VMEM_SHARED((_NPAD, _D_HID), jnp.float32),
        pltpu.SemaphoreType.DMA,
        pltpu.SemaphoreType.DMA,
    ],
)
def _agg_kernel(y0_hbm, y1_hbm, edge_hbm, s0_hbm, s1_hbm,
                idx_s, idx_d, rows_v, acc_sh, sem_a, sem_b):
    cid = lax.axis_index("c")
    sid = lax.axis_index("s")

    def icopy(y_ref):
        def body(k, carry):
            pltpu.sync_copy(y_ref.at[pl.ds(sid * 640 + k * 128, 128)],
                            acc_sh.at[pl.ds(sid * 640 + k * 128, 128)])
            return carry
        lax.fori_loop(0, 5, body, 0)

    @pl.when(cid == 0)
    def _():
        icopy(y0_hbm)

    @pl.when(cid == 1)
    def _():
        icopy(y1_hbm)

    plsc.subcore_barrier()

    def run(y_ref):
        buf = [rows_v.at[0, pl.ds(0, _CB)], rows_v.at[1, pl.ds(0, _CB)]]

        def stage(st, carry):
            pltpu.sync_copy(
                edge_hbm.at[0, sid, pl.ds(st * _AGG_STCH, _AGG_STCH)], idx_s)
            pltpu.sync_copy(
                edge_hbm.at[1, sid, pl.ds(st * _AGG_STCH, _AGG_STCH)], idx_d)
            pltpu.async_copy(y_ref.at[idx_s.at[0]], buf[0], sem_a)

            def body(jj, c2):
                ca = 2 * jj
                cb = 2 * jj + 1
                pltpu.async_copy(y_ref.at[idx_s.at[cb]], buf[1], sem_b)
                pltpu.make_async_copy(y_ref.at[idx_s.at[ca]], buf[0],
                                      sem_a).wait()
                pltpu.sync_copy(buf[0], acc_sh.at[idx_d.at[ca]], add=True)
                nxt = lax.rem(ca + 2, _AGG_STCH)
                pltpu.async_copy(y_ref.at[idx_s.at[nxt]], buf[0], sem_a)
                pltpu.make_async_copy(y_ref.at[idx_s.at[cb]], buf[1],
                                      sem_b).wait()
                pltpu.sync_copy(buf[1], acc_sh.at[idx_d.at[cb]], add=True)
                return c2

            lax.fori_loop(0, _AGG_STCH // 2, body, 0)
            pltpu.make_async_copy(y_ref.at[idx_s.at[0]], buf[0],
                                  sem_a).wait()
            return carry

        lax.fori_loop(0, _AGG_STAGES, stage, 0)

    @pl.when(cid == 0)
    def _():
        run(y0_hbm)

    @pl.when(cid == 1)
    def _():
        run(y1_hbm)

    plsc.subcore_barrier()

    @pl.when(cid == 0)
    def _():
        pltpu.sync_copy(acc_sh.at[pl.ds(sid * 640, 640)],
                        s0_hbm.at[pl.ds(sid * 640, 640)])

    @pl.when(cid == 1)
    def _():
        pltpu.sync_copy(acc_sh.at[pl.ds(sid * 640, 640)],
                        s1_hbm.at[pl.ds(sid * 640, 640)])


def _prep_body(cnt_ref, x_ref, y0_ref, y1_ref):
    deg = cnt_ref[:, 0] + cnt_ref[:, 1] + 1.0
    dinv = lax.rsqrt(deg)
    y0_ref[...] = x_ref[:, :_D_HID] * dinv[:, None]
    y1_ref[...] = x_ref[:, _D_HID:] * dinv[:, None]


_ROWB = 2000


def _prep_call(cnt_t, x):
    return pl.pallas_call(
        _prep_body,
        grid=(_N // _ROWB,),
        in_specs=[
            pl.BlockSpec((_ROWB, 2), lambda i: (i, 0)),
            pl.BlockSpec((_ROWB, _D_IN), lambda i: (i, 0)),
        ],
        out_specs=[
            pl.BlockSpec((_ROWB, _D_HID), lambda i: (i, 0)),
            pl.BlockSpec((_ROWB, _D_HID), lambda i: (i, 0)),
        ],
        out_shape=[
            jax.ShapeDtypeStruct((_NPAD, _D_HID), jnp.float32),
            jax.ShapeDtypeStruct((_NPAD, _D_HID), jnp.float32),
        ],
    )(cnt_t, x)


def _final_body(cnt_ref, s0_ref, s1_ref, a_ref, wcat_ref, bcat_ref,
                wf_ref, bf_ref, wc_ref, bc_ref, out_ref):
    deg = cnt_ref[:, 0] + cnt_ref[:, 1] + 1.0
    dinv = lax.rsqrt(deg)[:, None]
    agg = jnp.concatenate(
        [s0_ref[...] * dinv, s1_ref[...] * dinv], axis=1)
    hcat = jnp.maximum(
        jnp.dot(agg, wcat_ref[...], preferred_element_type=jnp.float32)
        + bcat_ref[...], 0.0)
    alpha = 1.0 - a_ref[...]
    h_fused = alpha * hcat[:, :_D_HID] + (1.0 - alpha) * hcat[:, _D_HID:]
    h = jnp.maximum(
        jnp.dot(h_fused, wf_ref[...], preferred_element_type=jnp.float32)
        + bf_ref[...], 0.0)
    out_ref[...] = (
        jnp.dot(h, wc_ref[...], preferred_element_type=jnp.float32)
        + bc_ref[...])


def _final_call(cnt_t, s0, s1, h_node2, wcat, bcat, wf, bf, wc, bc):
    nb = _N // _ROWB
    return pl.pallas_call(
        _final_body,
        grid=(nb,),
        in_specs=[
            pl.BlockSpec((_ROWB, 2), lambda i: (i, 0)),
            pl.BlockSpec((_ROWB, _D_HID), lambda i: (i, 0)),
            pl.BlockSpec((_ROWB, _D_HID), lambda i: (i, 0)),
            pl.BlockSpec((_ROWB, 1), lambda i: (i, 0)),
            pl.BlockSpec((_D_IN, _D_IN), lambda i: (0, 0)),
            pl.BlockSpec((1, _D_IN), lambda i: (0, 0)),
            pl.BlockSpec((_D_HID, 64), lambda i: (0, 0)),
            pl.BlockSpec((1, 64), lambda i: (0, 0)),
            pl.BlockSpec((64, 2), lambda i: (0, 0)),
            pl.BlockSpec((1, 2), lambda i: (0, 0)),
        ],
        out_specs=pl.BlockSpec((_ROWB, 2), lambda i: (i, 0)),
        out_shape=jax.ShapeDtypeStruct((_N, 2), jnp.float32),
    )(cnt_t, s0, s1, h_node2, wcat, bcat, wf, bf, wc, bc)


def kernel(x, edge_index, h_node, W_homo, b_homo, W_hetero, b_hetero,
           W_fusion, b_fusion, W_cls, b_cls):
    e4 = edge_index.reshape(2, _NS, _AGG_NCHUNK, _CB)
    cnt = _deg_kernel(e4)
    cnt_t = cnt.T[:_N]
    y0, y1 = _prep_call(cnt_t, x)
    s0, s1 = _agg_kernel(y0, y1, e4)
    wcat = jnp.concatenate([W_homo, W_hetero], axis=1)
    bcat = jnp.concatenate([b_homo, b_hetero]).reshape(1, _D_IN)
    logits = _final_call(cnt_t, s0, s1, h_node.reshape(_N, 1),
                         wcat, bcat, W_fusion, b_fusion.reshape(1, 64),
                         W_cls, b_cls.reshape(1, 2))
    return logits


# speedup vs baseline: 28.2134x; 1.0695x over previous
import functools

import jax
import jax.numpy as jnp
from jax import lax
from jax.experimental import pallas as pl
from jax.experimental.pallas import tpu as pltpu
from jax.experimental.pallas import tpu_sc as plsc

_N = 10000
_E = 160000
_NPAD = 10240
_D_IN = 256
_D_HID = 128

_NC = 2
_NS = 16
_CB = 125

_DEG_NCHUNK = 40
_AGG_NCHUNK = 80
_AGG_STAGES = 2
_AGG_STCH = _AGG_NCHUNK // _AGG_STAGES

_sc_mesh = plsc.VectorSubcoreMesh(core_axis_name="c", subcore_axis_name="s")


@functools.partial(
    pl.kernel,
    mesh=_sc_mesh,
    out_type=jax.ShapeDtypeStruct((_NC, _NPAD), jnp.float32),
    scratch_types=[
        pltpu.VMEM((_DEG_NCHUNK, _CB), jnp.int32),
        pltpu.VMEM((128,), jnp.float32),
        pltpu.VMEM((640,), jnp.float32),
        pltpu.VMEM_SHARED((_NPAD,), jnp.float32),
    ],
)
def _deg_kernel(edge_hbm, out_hbm, idx_v, ones_v, zeros_v, acc_sh):
    cid = lax.axis_index("c")
    sid = lax.axis_index("s")
    w = cid * _NS + sid
    for i in range(8):
        ones_v[pl.ds(i * 16, 16)] = jnp.ones((16,), jnp.float32)

    def zfill(k, carry):
        zeros_v[pl.ds(k * 16, 16)] = jnp.zeros((16,), jnp.float32)
        return carry

    lax.fori_loop(0, 40, zfill, 0)
    pltpu.sync_copy(zeros_v, acc_sh.at[pl.ds(sid * 640, 640)])
    pltpu.sync_copy(
        edge_hbm.at[1, w // 2, pl.ds((w % 2) * _DEG_NCHUNK, _DEG_NCHUNK)],
        idx_v)
    plsc.subcore_barrier()

    def body(j, carry):
        pltpu.sync_copy(ones_v.at[pl.ds(0, _CB)], acc_sh.at[idx_v.at[j]],
                        add=True)
        return carry

    lax.fori_loop(0, _DEG_NCHUNK, body, 0)
    plsc.subcore_barrier()
    pltpu.sync_copy(acc_sh.at[pl.ds(sid * 640, 640)],
                    out_hbm.at[cid, pl.ds(sid * 640, 640)])


@functools.partial(
    pl.kernel,
    mesh=_sc_mesh,
    out_type=[
        jax.ShapeDtypeStruct((_NPAD, _D_HID), jnp.float32),
        jax.ShapeDtypeStruct((_NPAD, _D_HID), jnp.float32),
    ],
    scratch_types=[
        pltpu.VMEM((_AGG_STCH, _CB), jnp.int32),
        pltpu.VMEM((_AGG_STCH, _CB), jnp.int32),
        pltpu.VMEM((2, 128, _D_HID), jnp.float32),
        pltpu.VMEM_SHARED((_NPAD, _D_HID), jnp.float32),
        pltpu.SemaphoreType.DMA,
        pltpu.SemaphoreType.DMA,
    ],
)
def _agg_kernel(y0_hbm, y1_hbm, edge_hbm, s0_hbm, s1_hbm,
                idx_s, idx_d, rows_v, acc_sh, sem_a, sem_b):
    cid = lax.axis_index("c")
    sid = lax.axis_index("s")

    def icopy(y_ref):
        def body(k, carry):
            pltpu.sync_copy(y_ref.at[pl.ds(sid * 640 + k * 128, 128)],
                            acc_sh.at[pl.ds(sid * 640 + k * 128, 128)])
            return carry
        lax.fori_loop(0, 5, body, 0)

    @pl.when(cid == 0)
    def _():
        icopy(y0_hbm)

    @pl.when(cid == 1)
    def _():
        icopy(y1_hbm)

    plsc.subcore_barrier()

    def run(y_ref):
        buf = [rows_v.at[0, pl.ds(0, _CB)], rows_v.at[1, pl.ds(0, _CB)]]

        def stage(st, carry):
            pltpu.sync_copy(
                edge_hbm.at[0, sid, pl.ds(st * _AGG_STCH, _AGG_STCH)], idx_s)
            pltpu.sync_copy(
                edge_hbm.at[1, sid, pl.ds(st * _AGG_STCH, _AGG_STCH)], idx_d)
            pltpu.async_copy(y_ref.at[idx_s.at[0]], buf[0], sem_a)

            def body(jj, c2):
                ca = 2 * jj
                cb = 2 * jj + 1
                pltpu.async_copy(y_ref.at[idx_s.at[cb]], buf[1], sem_b)
                pltpu.make_async_copy(y_ref.at[idx_s.at[ca]], buf[0],
                                      sem_a).wait()
                pltpu.sync_copy(buf[0], acc_sh.at[idx_d.at[ca]], add=True)

                @pl.when(ca + 2 < _AGG_STCH)
                def _():
                    pltpu.async_copy(y_ref.at[idx_s.at[ca + 2]], buf[0],
                                     sem_a)

                pltpu.make_async_copy(y_ref.at[idx_s.at[cb]], buf[1],
                                      sem_b).wait()
                pltpu.sync_copy(buf[1], acc_sh.at[idx_d.at[cb]], add=True)
                return c2

            lax.fori_loop(0, _AGG_STCH // 2, body, 0)
            return carry

        lax.fori_loop(0, _AGG_STAGES, stage, 0)

    @pl.when(cid == 0)
    def _():
        run(y0_hbm)

    @pl.when(cid == 1)
    def _():
        run(y1_hbm)

    plsc.subcore_barrier()

    @pl.when(cid == 0)
    def _():
        pltpu.sync_copy(acc_sh.at[pl.ds(sid * 640, 640)],
                        s0_hbm.at[pl.ds(sid * 640, 640)])

    @pl.when(cid == 1)
    def _():
        pltpu.sync_copy(acc_sh.at[pl.ds(sid * 640, 640)],
                        s1_hbm.at[pl.ds(sid * 640, 640)])


def _prep_body(cnt_ref, x_ref, y0_ref, y1_ref):
    deg = cnt_ref[:, 0] + cnt_ref[:, 1] + 1.0
    dinv = lax.rsqrt(deg)
    y0_ref[...] = x_ref[:, :_D_HID] * dinv[:, None]
    y1_ref[...] = x_ref[:, _D_HID:] * dinv[:, None]


_ROWB = 2000


def _prep_call(cnt_t, x):
    return pl.pallas_call(
        _prep_body,
        grid=(_N // _ROWB,),
        in_specs=[
            pl.BlockSpec((_ROWB, 2), lambda i: (i, 0)),
            pl.BlockSpec((_ROWB, _D_IN), lambda i: (i, 0)),
        ],
        out_specs=[
            pl.BlockSpec((_ROWB, _D_HID), lambda i: (i, 0)),
            pl.BlockSpec((_ROWB, _D_HID), lambda i: (i, 0)),
        ],
        out_shape=[
            jax.ShapeDtypeStruct((_NPAD, _D_HID), jnp.float32),
            jax.ShapeDtypeStruct((_NPAD, _D_HID), jnp.float32),
        ],
    )(cnt_t, x)


def _final_body(cnt_ref, s0_ref, s1_ref, a_ref, wcat_ref, bcat_ref,
                wf_ref, bf_ref, wc_ref, bc_ref, out_ref):
    deg = cnt_ref[:, 0] + cnt_ref[:, 1] + 1.0
    dinv = lax.rsqrt(deg)[:, None]
    agg = jnp.concatenate(
        [s0_ref[...] * dinv, s1_ref[...] * dinv], axis=1)
    hcat = jnp.maximum(
        jnp.dot(agg, wcat_ref[...], preferred_element_type=jnp.float32)
        + bcat_ref[...], 0.0)
    alpha = 1.0 - a_ref[...]
    h_fused = alpha * hcat[:, :_D_HID] + (1.0 - alpha) * hcat[:, _D_HID:]
    h = jnp.maximum(
        jnp.dot(h_fused, wf_ref[...], preferred_element_type=jnp.float32)
        + bf_ref[...], 0.0)
    out_ref[...] = (
        jnp.dot(h, wc_ref[...], preferred_element_type=jnp.float32)
        + bc_ref[...])


def _final_call(cnt_t, s0, s1, h_node2, wcat, bcat, wf, bf, wc, bc):
    nb = _N // _ROWB
    return pl.pallas_call(
        _final_body,
        grid=(nb,),
        in_specs=[
            pl.BlockSpec((_ROWB, 2), lambda i: (i, 0)),
            pl.BlockSpec((_ROWB, _D_HID), lambda i: (i, 0)),
            pl.BlockSpec((_ROWB, _D_HID), lambda i: (i, 0)),
            pl.BlockSpec((_ROWB, 1), lambda i: (i, 0)),
            pl.BlockSpec((_D_IN, _D_IN), lambda i: (0, 0)),
            pl.BlockSpec((1, _D_IN), lambda i: (0, 0)),
            pl.BlockSpec((_D_HID, 64), lambda i: (0, 0)),
            pl.BlockSpec((1, 64), lambda i: (0, 0)),
            pl.BlockSpec((64, 2), lambda i: (0, 0)),
            pl.BlockSpec((1, 2), lambda i: (0, 0)),
        ],
        out_specs=pl.BlockSpec((_ROWB, 2), lambda i: (i, 0)),
        out_shape=jax.ShapeDtypeStruct((_N, 2), jnp.float32),
    )(cnt_t, s0, s1, h_node2, wcat, bcat, wf, bf, wc, bc)


def kernel(x, edge_index, h_node, W_homo, b_homo, W_hetero, b_hetero,
           W_fusion, b_fusion, W_cls, b_cls):
    e4 = edge_index.reshape(2, _NS, _AGG_NCHUNK, _CB)
    cnt = _deg_kernel(e4)
    cnt_t = cnt.T[:_N]
    y0, y1 = _prep_call(cnt_t, x)
    s0, s1 = _agg_kernel(y0, y1, e4)
    wcat = jnp.concatenate([W_homo, W_hetero], axis=1)
    bcat = jnp.concatenate([b_homo, b_hetero]).reshape(1, _D_IN)
    logits = _final_call(cnt_t, s0, s1, h_node.reshape(_N, 1),
                         wcat, bcat, W_fusion, b_fusion.reshape(1, 64),
                         W_cls, b_cls.reshape(1, 2))
    return logits


# speedup vs baseline: 28.6980x; 1.0172x over previous
import functools

import jax
import jax.numpy as jnp
from jax import lax
from jax.experimental import pallas as pl
from jax.experimental.pallas import tpu as pltpu
from jax.experimental.pallas import tpu_sc as plsc

_N = 10000
_E = 160000
_NPAD = 10240
_D_IN = 256
_D_HID = 128

_NC = 2
_NS = 16
_CB = 125

_DEG_NCHUNK = 40
_AGG_NCHUNK = 80
_AGG_STAGES = 2
_AGG_STCH = _AGG_NCHUNK // _AGG_STAGES

_sc_mesh = plsc.VectorSubcoreMesh(core_axis_name="c", subcore_axis_name="s")


@functools.partial(
    pl.kernel,
    mesh=_sc_mesh,
    out_type=jax.ShapeDtypeStruct((_NC, _NPAD), jnp.float32),
    scratch_types=[
        pltpu.VMEM((_DEG_NCHUNK, _CB), jnp.int32),
        pltpu.VMEM((128,), jnp.float32),
        pltpu.VMEM((640,), jnp.float32),
        pltpu.VMEM_SHARED((_NPAD,), jnp.float32),
    ],
)
def _deg_kernel(edge_hbm, out_hbm, idx_v, ones_v, zeros_v, acc_sh):
    cid = lax.axis_index("c")
    sid = lax.axis_index("s")
    w = cid * _NS + sid
    for i in range(8):
        ones_v[pl.ds(i * 16, 16)] = jnp.ones((16,), jnp.float32)

    def zfill(k, carry):
        zeros_v[pl.ds(k * 16, 16)] = jnp.zeros((16,), jnp.float32)
        return carry

    lax.fori_loop(0, 40, zfill, 0)
    pltpu.sync_copy(zeros_v, acc_sh.at[pl.ds(sid * 640, 640)])
    pltpu.sync_copy(
        edge_hbm.at[1, w // 2, pl.ds((w % 2) * _DEG_NCHUNK, _DEG_NCHUNK)],
        idx_v)
    plsc.subcore_barrier()

    def body(j, carry):
        pltpu.sync_copy(ones_v.at[pl.ds(0, _CB)], acc_sh.at[idx_v.at[j]],
                        add=True)
        return carry

    lax.fori_loop(0, _DEG_NCHUNK, body, 0)
    plsc.subcore_barrier()
    pltpu.sync_copy(acc_sh.at[pl.ds(sid * 640, 640)],
                    out_hbm.at[cid, pl.ds(sid * 640, 640)])


@functools.partial(
    pl.kernel,
    mesh=_sc_mesh,
    out_type=[
        jax.ShapeDtypeStruct((_NPAD, _D_HID), jnp.float32),
        jax.ShapeDtypeStruct((_NPAD, _D_HID), jnp.float32),
    ],
    scratch_types=[
        pltpu.VMEM((_AGG_STCH, _CB), jnp.int32),
        pltpu.VMEM((_AGG_STCH, _CB), jnp.int32),
        pltpu.VMEM((2, 128, _D_HID), jnp.float32),
        pltpu.VMEM_SHARED((_NPAD, _D_HID), jnp.float32),
        pltpu.SemaphoreType.DMA,
        pltpu.SemaphoreType.DMA,
    ],
)
def _agg_kernel(y0_hbm, y1_hbm, edge_hbm, s0_hbm, s1_hbm,
                idx_s, idx_d, rows_v, acc_sh, sem_a, sem_b):
    cid = lax.axis_index("c")
    sid = lax.axis_index("s")

    def icopy(y_ref):
        def body(k, carry):
            pltpu.sync_copy(y_ref.at[pl.ds(sid * 640 + k * 128, 128)],
                            acc_sh.at[pl.ds(sid * 640 + k * 128, 128)])
            return carry
        lax.fori_loop(0, 5, body, 0)

    @pl.when(cid == 0)
    def _():
        icopy(y0_hbm)

    @pl.when(cid == 1)
    def _():
        icopy(y1_hbm)

    plsc.subcore_barrier()

    def run(y_ref):
        buf = [rows_v.at[0, pl.ds(0, _CB)], rows_v.at[1, pl.ds(0, _CB)]]

        def stage(st, carry):
            pltpu.sync_copy(
                edge_hbm.at[0, sid, pl.ds(st * _AGG_STCH, _AGG_STCH)], idx_s)
            pltpu.sync_copy(
                edge_hbm.at[1, sid, pl.ds(st * _AGG_STCH, _AGG_STCH)], idx_d)
            pltpu.async_copy(y_ref.at[idx_s.at[0]], buf[0], sem_a)

            def body(jj, c2):
                ca = 2 * jj
                cb = 2 * jj + 1
                pltpu.async_copy(y_ref.at[idx_s.at[cb]], buf[1], sem_b)
                pltpu.make_async_copy(y_ref.at[idx_s.at[ca]], buf[0],
                                      sem_a).wait()
                pltpu.sync_copy(buf[0], acc_sh.at[idx_d.at[ca]], add=True)

                @pl.when(ca + 2 < _AGG_STCH)
                def _():
                    pltpu.async_copy(y_ref.at[idx_s.at[ca + 2]], buf[0],
                                     sem_a)

                pltpu.make_async_copy(y_ref.at[idx_s.at[cb]], buf[1],
                                      sem_b).wait()
                pltpu.sync_copy(buf[1], acc_sh.at[idx_d.at[cb]], add=True)
                return c2

            lax.fori_loop(0, _AGG_STCH // 2, body, 0)
            return carry

        lax.fori_loop(0, _AGG_STAGES, stage, 0)

    @pl.when(cid == 0)
    def _():
        run(y0_hbm)

    @pl.when(cid == 1)
    def _():
        run(y1_hbm)

    plsc.subcore_barrier()

    @pl.when(cid == 0)
    def _():
        pltpu.sync_copy(acc_sh.at[pl.ds(sid * 640, 640)],
                        s0_hbm.at[pl.ds(sid * 640, 640)])

    @pl.when(cid == 1)
    def _():
        pltpu.sync_copy(acc_sh.at[pl.ds(sid * 640, 640)],
                        s1_hbm.at[pl.ds(sid * 640, 640)])


def _prep_body(cnt_ref, x_ref, y0_ref, y1_ref):
    deg = cnt_ref[:, 0] + cnt_ref[:, 1] + 1.0
    dinv = lax.rsqrt(deg)
    y0_ref[...] = x_ref[:, :_D_HID] * dinv[:, None]
    y1_ref[...] = x_ref[:, _D_HID:] * dinv[:, None]


_ROWB = 5000


def _prep_call(cnt_t, x):
    return pl.pallas_call(
        _prep_body,
        grid=(_N // _ROWB,),
        in_specs=[
            pl.BlockSpec((_ROWB, 2), lambda i: (i, 0)),
            pl.BlockSpec((_ROWB, _D_IN), lambda i: (i, 0)),
        ],
        out_specs=[
            pl.BlockSpec((_ROWB, _D_HID), lambda i: (i, 0)),
            pl.BlockSpec((_ROWB, _D_HID), lambda i: (i, 0)),
        ],
        out_shape=[
            jax.ShapeDtypeStruct((_NPAD, _D_HID), jnp.float32),
            jax.ShapeDtypeStruct((_NPAD, _D_HID), jnp.float32),
        ],
    )(cnt_t, x)


def _final_body(cnt_ref, s0_ref, s1_ref, a_ref, wcat_ref, bcat_ref,
                wf_ref, bf_ref, wc_ref, bc_ref, out_ref):
    deg = cnt_ref[:, 0] + cnt_ref[:, 1] + 1.0
    dinv = lax.rsqrt(deg)[:, None]
    agg = jnp.concatenate(
        [s0_ref[...] * dinv, s1_ref[...] * dinv], axis=1)
    hcat = jnp.maximum(
        jnp.dot(agg, wcat_ref[...], preferred_element_type=jnp.float32)
        + bcat_ref[...], 0.0)
    alpha = 1.0 - a_ref[...]
    h_fused = alpha * hcat[:, :_D_HID] + (1.0 - alpha) * hcat[:, _D_HID:]
    h = jnp.maximum(
        jnp.dot(h_fused, wf_ref[...], preferred_element_type=jnp.float32)
        + bf_ref[...], 0.0)
    out_ref[...] = (
        jnp.dot(h, wc_ref[...], preferred_element_type=jnp.float32)
        + bc_ref[...])


def _final_call(cnt_t, s0, s1, h_node2, wcat, bcat, wf, bf, wc, bc):
    nb = _N // _ROWB
    return pl.pallas_call(
        _final_body,
        grid=(nb,),
        in_specs=[
            pl.BlockSpec((_ROWB, 2), lambda i: (i, 0)),
            pl.BlockSpec((_ROWB, _D_HID), lambda i: (i, 0)),
            pl.BlockSpec((_ROWB, _D_HID), lambda i: (i, 0)),
            pl.BlockSpec((_ROWB, 1), lambda i: (i, 0)),
            pl.BlockSpec((_D_IN, _D_IN), lambda i: (0, 0)),
            pl.BlockSpec((1, _D_IN), lambda i: (0, 0)),
            pl.BlockSpec((_D_HID, 64), lambda i: (0, 0)),
            pl.BlockSpec((1, 64), lambda i: (0, 0)),
            pl.BlockSpec((64, 2), lambda i: (0, 0)),
            pl.BlockSpec((1, 2), lambda i: (0, 0)),
        ],
        out_specs=pl.BlockSpec((_ROWB, 2), lambda i: (i, 0)),
        out_shape=jax.ShapeDtypeStruct((_N, 2), jnp.float32),
    )(cnt_t, s0, s1, h_node2, wcat, bcat, wf, bf, wc, bc)


def kernel(x, edge_index, h_node, W_homo, b_homo, W_hetero, b_hetero,
           W_fusion, b_fusion, W_cls, b_cls):
    e4 = edge_index.reshape(2, _NS, _AGG_NCHUNK, _CB)
    cnt = _deg_kernel(e4)
    cnt_t = cnt.T[:_N]
    y0, y1 = _prep_call(cnt_t, x)
    s0, s1 = _agg_kernel(y0, y1, e4)
    wcat = jnp.concatenate([W_homo, W_hetero], axis=1)
    bcat = jnp.concatenate([b_homo, b_hetero]).reshape(1, _D_IN)
    logits = _final_call(cnt_t, s0, s1, h_node.reshape(_N, 1),
                         wcat, bcat, W_fusion, b_fusion.reshape(1, 64),
                         W_cls, b_cls.reshape(1, 2))
    return logits


# speedup vs baseline: 29.3384x; 1.0223x over previous
import functools

import jax
import jax.numpy as jnp
from jax import lax
from jax.experimental import pallas as pl
from jax.experimental.pallas import tpu as pltpu
from jax.experimental.pallas import tpu_sc as plsc

_N = 10000
_E = 160000
_NPAD = 10240
_D_IN = 256
_D_HID = 128

_NC = 2
_NS = 16
_CB = 125

_DEG_NCHUNK = 40
_AGG_NCHUNK = 80
_AGG_STAGES = 2
_AGG_STCH = _AGG_NCHUNK // _AGG_STAGES

_sc_mesh = plsc.VectorSubcoreMesh(core_axis_name="c", subcore_axis_name="s")


@functools.partial(
    pl.kernel,
    mesh=_sc_mesh,
    out_type=jax.ShapeDtypeStruct((_NC, _NPAD), jnp.float32),
    scratch_types=[
        pltpu.VMEM((_DEG_NCHUNK, _CB), jnp.int32),
        pltpu.VMEM((128,), jnp.float32),
        pltpu.VMEM((640,), jnp.float32),
        pltpu.VMEM_SHARED((_NPAD,), jnp.float32),
    ],
)
def _deg_kernel(edge_hbm, out_hbm, idx_v, ones_v, zeros_v, acc_sh):
    cid = lax.axis_index("c")
    sid = lax.axis_index("s")
    w = cid * _NS + sid
    for i in range(8):
        ones_v[pl.ds(i * 16, 16)] = jnp.ones((16,), jnp.float32)

    def zfill(k, carry):
        zeros_v[pl.ds(k * 16, 16)] = jnp.zeros((16,), jnp.float32)
        return carry

    lax.fori_loop(0, 40, zfill, 0)
    pltpu.sync_copy(zeros_v, acc_sh.at[pl.ds(sid * 640, 640)])
    pltpu.sync_copy(
        edge_hbm.at[1, w // 2, pl.ds((w % 2) * _DEG_NCHUNK, _DEG_NCHUNK)],
        idx_v)
    plsc.subcore_barrier()

    def body(j, carry):
        pltpu.sync_copy(ones_v.at[pl.ds(0, _CB)], acc_sh.at[idx_v.at[j]],
                        add=True)
        return carry

    lax.fori_loop(0, _DEG_NCHUNK, body, 0)
    plsc.subcore_barrier()
    pltpu.sync_copy(acc_sh.at[pl.ds(sid * 640, 640)],
                    out_hbm.at[cid, pl.ds(sid * 640, 640)])


@functools.partial(
    pl.kernel,
    mesh=_sc_mesh,
    out_type=[
        jax.ShapeDtypeStruct((_NPAD, _D_HID), jnp.float32),
        jax.ShapeDtypeStruct((_NPAD, _D_HID), jnp.float32),
    ],
    scratch_types=[
        pltpu.VMEM((_AGG_STCH, _CB), jnp.int32),
        pltpu.VMEM((_AGG_STCH, _CB), jnp.int32),
        pltpu.VMEM((2, 128, _D_HID), jnp.float32),
        pltpu.VMEM_SHARED((_NPAD, _D_HID), jnp.float32),
        pltpu.SemaphoreType.DMA,
        pltpu.SemaphoreType.DMA,
    ],
)
def _agg_kernel(y0_hbm, y1_hbm, edge_hbm, s0_hbm, s1_hbm,
                idx_s, idx_d, rows_v, acc_sh, sem_a, sem_b):
    cid = lax.axis_index("c")
    sid = lax.axis_index("s")

    def icopy(y_ref):
        def body(k, carry):
            pltpu.sync_copy(y_ref.at[pl.ds(sid * 640 + k * 128, 128)],
                            acc_sh.at[pl.ds(sid * 640 + k * 128, 128)])
            return carry
        lax.fori_loop(0, 5, body, 0)

    @pl.when(cid == 0)
    def _():
        icopy(y0_hbm)

    @pl.when(cid == 1)
    def _():
        icopy(y1_hbm)

    plsc.subcore_barrier()

    def run(y_ref):
        buf = [rows_v.at[0, pl.ds(0, _CB)], rows_v.at[1, pl.ds(0, _CB)]]

        def stage(st, carry):
            pltpu.sync_copy(
                edge_hbm.at[0, sid, pl.ds(st * _AGG_STCH, _AGG_STCH)], idx_s)
            pltpu.sync_copy(
                edge_hbm.at[1, sid, pl.ds(st * _AGG_STCH, _AGG_STCH)], idx_d)
            pltpu.async_copy(y_ref.at[idx_s.at[0]], buf[0], sem_a)

            def body(jj, c2):
                ca = 2 * jj
                cb = 2 * jj + 1
                pltpu.async_copy(y_ref.at[idx_s.at[cb]], buf[1], sem_b)
                pltpu.make_async_copy(y_ref.at[idx_s.at[ca]], buf[0],
                                      sem_a).wait()
                pltpu.sync_copy(buf[0], acc_sh.at[idx_d.at[ca]], add=True)

                @pl.when(ca + 2 < _AGG_STCH)
                def _():
                    pltpu.async_copy(y_ref.at[idx_s.at[ca + 2]], buf[0],
                                     sem_a)

                pltpu.make_async_copy(y_ref.at[idx_s.at[cb]], buf[1],
                                      sem_b).wait()
                pltpu.sync_copy(buf[1], acc_sh.at[idx_d.at[cb]], add=True)
                return c2

            lax.fori_loop(0, _AGG_STCH // 2, body, 0)
            return carry

        lax.fori_loop(0, _AGG_STAGES, stage, 0)

    @pl.when(cid == 0)
    def _():
        run(y0_hbm)

    @pl.when(cid == 1)
    def _():
        run(y1_hbm)

    plsc.subcore_barrier()

    @pl.when(cid == 0)
    def _():
        pltpu.sync_copy(acc_sh.at[pl.ds(sid * 640, 640)],
                        s0_hbm.at[pl.ds(sid * 640, 640)])

    @pl.when(cid == 1)
    def _():
        pltpu.sync_copy(acc_sh.at[pl.ds(sid * 640, 640)],
                        s1_hbm.at[pl.ds(sid * 640, 640)])


def _prep_body(cnt_ref, x_ref, y0_ref, y1_ref, dinv_ref):
    deg = cnt_ref[0, :] + cnt_ref[1, :] + 1.0
    dinv = lax.rsqrt(deg)
    dinv_ref[...] = dinv[:, None]
    dcol = dinv[:_N, None]
    y0_ref[pl.ds(0, _N), :] = x_ref[:, :_D_HID] * dcol
    y1_ref[pl.ds(0, _N), :] = x_ref[:, _D_HID:] * dcol


_ROWB = 5000


def _prep_call(cnt, x):
    return pl.pallas_call(
        _prep_body,
        grid=(1,),
        in_specs=[
            pl.BlockSpec((2, _NPAD), lambda i: (0, 0)),
            pl.BlockSpec((_N, _D_IN), lambda i: (0, 0)),
        ],
        out_specs=[
            pl.BlockSpec((_NPAD, _D_HID), lambda i: (0, 0)),
            pl.BlockSpec((_NPAD, _D_HID), lambda i: (0, 0)),
            pl.BlockSpec((_NPAD, 1), lambda i: (0, 0)),
        ],
        out_shape=[
            jax.ShapeDtypeStruct((_NPAD, _D_HID), jnp.float32),
            jax.ShapeDtypeStruct((_NPAD, _D_HID), jnp.float32),
            jax.ShapeDtypeStruct((_NPAD, 1), jnp.float32),
        ],
    )(cnt, x)


def _final_body(dinv_ref, s0_ref, s1_ref, a_ref, wcat_ref, bcat_ref,
                wf_ref, bf_ref, wc_ref, bc_ref, out_ref):
    dinv = dinv_ref[...]
    agg = jnp.concatenate(
        [s0_ref[...] * dinv, s1_ref[...] * dinv], axis=1)
    hcat = jnp.maximum(
        jnp.dot(agg, wcat_ref[...], preferred_element_type=jnp.float32)
        + bcat_ref[...], 0.0)
    alpha = 1.0 - a_ref[...]
    h_fused = alpha * hcat[:, :_D_HID] + (1.0 - alpha) * hcat[:, _D_HID:]
    h = jnp.maximum(
        jnp.dot(h_fused, wf_ref[...], preferred_element_type=jnp.float32)
        + bf_ref[...], 0.0)
    out_ref[...] = (
        jnp.dot(h, wc_ref[...], preferred_element_type=jnp.float32)
        + bc_ref[...])


def _final_call(dinv2, s0, s1, h_node2, wcat, bcat, wf, bf, wc, bc):
    nb = _N // _ROWB
    return pl.pallas_call(
        _final_body,
        grid=(nb,),
        in_specs=[
            pl.BlockSpec((_ROWB, 1), lambda i: (i, 0)),
            pl.BlockSpec((_ROWB, _D_HID), lambda i: (i, 0)),
            pl.BlockSpec((_ROWB, _D_HID), lambda i: (i, 0)),
            pl.BlockSpec((_ROWB, 1), lambda i: (i, 0)),
            pl.BlockSpec((_D_IN, _D_IN), lambda i: (0, 0)),
            pl.BlockSpec((1, _D_IN), lambda i: (0, 0)),
            pl.BlockSpec((_D_HID, 64), lambda i: (0, 0)),
            pl.BlockSpec((1, 64), lambda i: (0, 0)),
            pl.BlockSpec((64, 2), lambda i: (0, 0)),
            pl.BlockSpec((1, 2), lambda i: (0, 0)),
        ],
        out_specs=pl.BlockSpec((_ROWB, 2), lambda i: (i, 0)),
        out_shape=jax.ShapeDtypeStruct((_N, 2), jnp.float32),
    )(dinv2, s0, s1, h_node2, wcat, bcat, wf, bf, wc, bc)


def kernel(x, edge_index, h_node, W_homo, b_homo, W_hetero, b_hetero,
           W_fusion, b_fusion, W_cls, b_cls):
    e4 = edge_index.reshape(2, _NS, _AGG_NCHUNK, _CB)
    cnt = _deg_kernel(e4)
    y0, y1, dinv2 = _prep_call(cnt, x)
    s0, s1 = _agg_kernel(y0, y1, e4)
    wcat = jnp.concatenate([W_homo, W_hetero], axis=1)
    bcat = jnp.concatenate([b_homo, b_hetero]).reshape(1, _D_IN)
    logits = _final_call(dinv2, s0, s1, h_node.reshape(_N, 1),
                         wcat, bcat, W_fusion, b_fusion.reshape(1, 64),
                         W_cls, b_cls.reshape(1, 2))
    return logits
